# baseline (speedup 1.0000x reference)
import functools

import jax
import jax.numpy as jnp
from jax import lax
from jax.experimental import pallas as pl
from jax.experimental.pallas import tpu as pltpu
from jax.experimental.pallas import tpu_sc as plsc

N = 10000
E = 640000
H = 128
G = 16
NSIG = int(N * 0.9 // G)

NC, NS = 2, 16
NW = NC * NS
CH = 128
NCHUNK = 160
EPW = NCHUNK * CH
EPAD = EPW * NW
DUMP = N
NP = 10112
SLAB = NP // NS
RB = 1000
GRID = N // RB


def _deg_body(dst_hbm, ones_hbm, zeros_hbm, out_hbm, idx_v, ones_v, acc_sh):
    cid = lax.axis_index("c")
    sid = lax.axis_index("s")
    wid = sid * NC + cid
    base = wid * EPW
    pltpu.sync_copy(zeros_hbm, acc_sh.at[pl.ds(sid * SLAB, SLAB)])
    pltpu.sync_copy(ones_hbm, ones_v)
    plsc.subcore_barrier()

    def step(i, carry):
        pltpu.sync_copy(dst_hbm.at[pl.ds(base + i * CH, CH)], idx_v)
        pltpu.sync_copy(ones_v, acc_sh.at[idx_v], add=True)
        return carry

    lax.fori_loop(0, NCHUNK, step, 0)
    plsc.subcore_barrier()
    pltpu.sync_copy(acc_sh.at[pl.ds(sid * SLAB, SLAB)],
                    out_hbm.at[cid, pl.ds(sid * SLAB, SLAB)])


def _spmm_body(src_hbm, dst_hbm, hws_hbm, zeros_hbm, out_hbm,
               sidx0, sidx1, didx0, didx1, rows0, rows1,
               acc_sh, gsem0, gsem1):
    cid = lax.axis_index("c")
    sid = lax.axis_index("s")
    wid = sid * NC + cid
    base = wid * EPW
    pltpu.sync_copy(zeros_hbm, acc_sh.at[pl.ds(sid * SLAB, SLAB)])

    pltpu.sync_copy(src_hbm.at[pl.ds(base, CH)], sidx0)
    pltpu.sync_copy(dst_hbm.at[pl.ds(base, CH)], didx0)
    pltpu.async_copy(hws_hbm.at[sidx0], rows0, gsem0)
    plsc.subcore_barrier()

    def step(j, carry):
        i0 = 2 * j
        pltpu.sync_copy(src_hbm.at[pl.ds(base + (i0 + 1) * CH, CH)], sidx1)
        pltpu.sync_copy(dst_hbm.at[pl.ds(base + (i0 + 1) * CH, CH)], didx1)
        pltpu.async_copy(hws_hbm.at[sidx1], rows1, gsem1)
        pltpu.make_async_copy(hws_hbm.at[sidx0], rows0, gsem0).wait()
        pltpu.sync_copy(rows0, acc_sh.at[didx0], add=True)
        pltpu.sync_copy(src_hbm.at[pl.ds(base + (i0 + 2) * CH, CH)], sidx0)
        pltpu.sync_copy(dst_hbm.at[pl.ds(base + (i0 + 2) * CH, CH)], didx0)
        pltpu.async_copy(hws_hbm.at[sidx0], rows0, gsem0)
        pltpu.make_async_copy(hws_hbm.at[sidx1], rows1, gsem1).wait()
        pltpu.sync_copy(rows1, acc_sh.at[didx1], add=True)
        return carry

    lax.fori_loop(0, NCHUNK // 2 - 1, step, 0)
    pltpu.sync_copy(src_hbm.at[pl.ds(base + (NCHUNK - 1) * CH, CH)], sidx1)
    pltpu.sync_copy(dst_hbm.at[pl.ds(base + (NCHUNK - 1) * CH, CH)], didx1)
    pltpu.async_copy(hws_hbm.at[sidx1], rows1, gsem1)
    pltpu.make_async_copy(hws_hbm.at[sidx0], rows0, gsem0).wait()
    pltpu.sync_copy(rows0, acc_sh.at[didx0], add=True)
    pltpu.make_async_copy(hws_hbm.at[sidx1], rows1, gsem1).wait()
    pltpu.sync_copy(rows1, acc_sh.at[didx1], add=True)
    plsc.subcore_barrier()
    pltpu.sync_copy(acc_sh.at[pl.ds(sid * SLAB, SLAB)],
                    out_hbm.at[cid, pl.ds(sid * SLAB, SLAB)])


@functools.cache
def _build_sc():
    mesh = plsc.VectorSubcoreMesh(core_axis_name="c", subcore_axis_name="s",
                                  num_cores=NC, num_subcores=NS)
    deg_k = pl.kernel(
        _deg_body,
        out_type=jax.ShapeDtypeStruct((NC, NP, H), jnp.float32),
        mesh=mesh,
        scratch_types=[
            pltpu.VMEM((CH,), jnp.int32),
            pltpu.VMEM((CH, H), jnp.float32),
            pltpu.VMEM_SHARED((NP, H), jnp.float32),
        ],
    )
    spmm_k = pl.kernel(
        _spmm_body,
        out_type=jax.ShapeDtypeStruct((NC, NP, H), jnp.float32),
        mesh=mesh,
        scratch_types=[
            pltpu.VMEM((CH,), jnp.int32),
            pltpu.VMEM((CH,), jnp.int32),
            pltpu.VMEM((CH,), jnp.int32),
            pltpu.VMEM((CH,), jnp.int32),
            pltpu.VMEM((CH, H), jnp.float32),
            pltpu.VMEM((CH, H), jnp.float32),
            pltpu.VMEM_SHARED((NP, H), jnp.float32),
            pltpu.SemaphoreType.DMA,
            pltpu.SemaphoreType.DMA,
        ],
    )
    return deg_k, spmm_k


def _sc_degree(dst1, onesH, zerosH):
    return _build_sc()[0](dst1, onesH, zerosH)


def _sc_spmm(src1, dst1, hws, zerosH):
    return _build_sc()[1](src1, dst1, hws, zerosH)


def _dinv_block(deg_ref):
    deg = deg_ref[0, :, 0:1] + deg_ref[1, :, 0:1] + 1.0
    return lax.rsqrt(deg)


def _enc_body(x_ref, deg_ref, sw_ref, sb_ref, tw_ref, tb_ref, w1_ref, o_ref):
    i = pl.program_id(0)
    rowid = lax.broadcasted_iota(jnp.int32, (RB, 1), 0) + i * RB
    is_sig = rowid < NSIG
    w = jnp.where(is_sig, sw_ref[...], tw_ref[...])
    b = jnp.where(is_sig, sb_ref[...], tb_ref[...])
    h0 = jnp.maximum(x_ref[...] * w + b, 0.0)
    hw = jnp.dot(h0, w1_ref[...], preferred_element_type=jnp.float32)
    o_ref[...] = hw * _dinv_block(deg_ref)


def _tc_encode(x, deg, sw, sb, tw, tb, w1):
    return pl.pallas_call(
        _enc_body,
        grid=(GRID,),
        in_specs=[
            pl.BlockSpec((RB, 1), lambda i: (i, 0)),
            pl.BlockSpec((NC, RB, H), lambda i: (0, i, 0)),
            pl.BlockSpec((1, H), lambda i: (0, 0)),
            pl.BlockSpec((1, H), lambda i: (0, 0)),
            pl.BlockSpec((1, H), lambda i: (0, 0)),
            pl.BlockSpec((1, H), lambda i: (0, 0)),
            pl.BlockSpec((H, H), lambda i: (0, 0)),
        ],
        out_specs=pl.BlockSpec((RB, H), lambda i: (i, 0)),
        out_shape=jax.ShapeDtypeStruct((N, H), jnp.float32),
    )(x, deg, sw, sb, tw, tb, w1)


def _mid_body(acc_ref, hws_ref, deg_ref, b1_ref, w2_ref, o_ref):
    dinv = _dinv_block(deg_ref)
    m = acc_ref[0] + acc_ref[1] + hws_ref[...]
    g = jnp.maximum(m * dinv + b1_ref[...], 0.0)
    o_ref[...] = jnp.dot(g, w2_ref[...], preferred_element_type=jnp.float32) * dinv


def _tc_mid(acc, hws, deg, b1, w2):
    return pl.pallas_call(
        _mid_body,
        grid=(GRID,),
        in_specs=[
            pl.BlockSpec((NC, RB, H), lambda i: (0, i, 0)),
            pl.BlockSpec((RB, H), lambda i: (i, 0)),
            pl.BlockSpec((NC, RB, H), lambda i: (0, i, 0)),
            pl.BlockSpec((1, H), lambda i: (0, 0)),
            pl.BlockSpec((H, H), lambda i: (0, 0)),
        ],
        out_specs=pl.BlockSpec((RB, H), lambda i: (i, 0)),
        out_shape=jax.ShapeDtypeStruct((N, H), jnp.float32),
    )(acc, hws, deg, b1, w2)


def _final_body(acc_ref, hws_ref, deg_ref, b2_ref, batch_ref,
                w1_ref, b1_ref, w2_ref, bb2_ref, w3_ref, b3_ref,
                o_ref, sums, counts):
    i = pl.program_id(0)

    @pl.when(i == 0)
    def _():
        sums[...] = jnp.zeros_like(sums)
        counts[...] = jnp.zeros_like(counts)

    dinv = _dinv_block(deg_ref)
    h2 = (acc_ref[0] + acc_ref[1] + hws_ref[...]) * dinv + b2_ref[...]
    oh = (batch_ref[...] == lax.broadcasted_iota(jnp.int32, (RB, G), 1))
    oh = oh.astype(jnp.float32)
    dn = (((0,), (0,)), ((), ()))
    sums[...] += lax.dot_general(oh, h2, dn, preferred_element_type=jnp.float32)
    counts[...] += lax.dot_general(oh, jnp.ones((RB, H), jnp.float32), dn,
                                   preferred_element_type=jnp.float32)

    @pl.when(i == GRID - 1)
    def _():
        pooled = sums[...] / jnp.maximum(counts[...], 1.0)
        z = jnp.maximum(jnp.dot(pooled, w1_ref[...],
                                preferred_element_type=jnp.float32) + b1_ref[...], 0.0)
        z = jnp.maximum(jnp.dot(z, w2_ref[...],
                                preferred_element_type=jnp.float32) + bb2_ref[...], 0.0)
        o_ref[...] = jnp.dot(z, w3_ref[...],
                             preferred_element_type=jnp.float32) + b3_ref[...]


def _tc_final(acc, hws, deg, b2, batch2d, cw1, cb1, cw2, cb2, cw3, cb3):
    return pl.pallas_call(
        _final_body,
        grid=(GRID,),
        in_specs=[
            pl.BlockSpec((NC, RB, H), lambda i: (0, i, 0)),
            pl.BlockSpec((RB, H), lambda i: (i, 0)),
            pl.BlockSpec((NC, RB, H), lambda i: (0, i, 0)),
            pl.BlockSpec((1, H), lambda i: (0, 0)),
            pl.BlockSpec((RB, 1), lambda i: (i, 0)),
            pl.BlockSpec((H, H), lambda i: (0, 0)),
            pl.BlockSpec((1, H), lambda i: (0, 0)),
            pl.BlockSpec((H, H // 2), lambda i: (0, 0)),
            pl.BlockSpec((1, H // 2), lambda i: (0, 0)),
            pl.BlockSpec((H // 2, 4), lambda i: (0, 0)),
            pl.BlockSpec((1, 4), lambda i: (0, 0)),
        ],
        out_specs=pl.BlockSpec((G, 4), lambda i: (0, 0)),
        out_shape=jax.ShapeDtypeStruct((G, 4), jnp.float32),
        scratch_shapes=[
            pltpu.VMEM((G, H), jnp.float32),
            pltpu.VMEM((G, H), jnp.float32),
        ],
    )(acc, hws, deg, b2, batch2d, cw1, cb1, cw2, cb2, cw3, cb3)


def kernel(x, edge_index, batch, sig_W, sig_b, ts_W, ts_b,
           gcn_W1, gcn_b1, gcn_W2, gcn_b2,
           c_W1, c_b1, c_W2, c_b2, c_W3, c_b3):
    npad = EPAD - E
    src1 = jnp.concatenate([edge_index[0], jnp.zeros((npad,), jnp.int32)])
    dst1 = jnp.concatenate([edge_index[1], jnp.full((npad,), DUMP, jnp.int32)])
    onesH = jnp.ones((CH, H), jnp.float32)
    zerosH = jnp.zeros((SLAB, H), jnp.float32)

    deg = _sc_degree(dst1, onesH, zerosH)
    hws1 = _tc_encode(x, deg, sig_W.reshape(1, H), sig_b.reshape(1, H),
                      ts_W.reshape(1, H), ts_b.reshape(1, H), gcn_W1)
    acc1 = _sc_spmm(src1, dst1, hws1, zerosH)
    hws2 = _tc_mid(acc1, hws1, deg, gcn_b1.reshape(1, H), gcn_W2)
    acc2 = _sc_spmm(src1, dst1, hws2, zerosH)
    return _tc_final(acc2, hws2, deg, gcn_b2.reshape(1, H),
                     batch.reshape(N, 1), c_W1, c_b1.reshape(1, H),
                     c_W2, c_b2.reshape(1, H // 2), c_W3, c_b3.reshape(1, 4))


# speedup vs baseline: 1.5353x; 1.5353x over previous
import functools

import jax
import jax.numpy as jnp
from jax import lax
from jax.experimental import pallas as pl
from jax.experimental.pallas import tpu as pltpu
from jax.experimental.pallas import tpu_sc as plsc

N = 10000
E = 640000
H = 128
G = 16
NSIG = int(N * 0.9 // G)

NC, NS = 2, 16
NW = NC * NS
CH = 128
NCHUNK = 158
EPW = NCHUNK * CH
EPAD = EPW * NW
DUMP = N
NP = 10240
SLAB = NP // NS
ZR = 128
RB = 1000
GRID = N // RB


def _deg_body(dst_hbm, ones_hbm, zeros_hbm, out_hbm, idx_v, ones_v, acc_sh):
    cid = lax.axis_index("c")
    sid = lax.axis_index("s")
    wid = sid * NC + cid
    base = wid * EPW
    for j in range(SLAB // ZR):
        pltpu.sync_copy(zeros_hbm, acc_sh.at[pl.ds(sid * SLAB + j * ZR, ZR)])
    pltpu.sync_copy(ones_hbm, ones_v)
    plsc.subcore_barrier()

    def step(i, carry):
        pltpu.sync_copy(dst_hbm.at[pl.ds(base + i * CH, CH)], idx_v)
        pltpu.sync_copy(ones_v, acc_sh.at[idx_v], add=True)
        return carry

    lax.fori_loop(0, NCHUNK, step, 0)
    plsc.subcore_barrier()
    pltpu.sync_copy(acc_sh.at[pl.ds(sid * SLAB, SLAB)],
                    out_hbm.at[cid, pl.ds(sid * SLAB, SLAB)])


def _spmm_body(src_hbm, dst_hbm, hws_hbm, zeros_hbm, out_hbm,
               sidx0, sidx1, didx0, didx1, rows0, rows1,
               acc_sh, gsem0, gsem1):
    cid = lax.axis_index("c")
    sid = lax.axis_index("s")
    wid = sid * NC + cid
    base = wid * EPW
    for j in range(SLAB // ZR):
        pltpu.sync_copy(zeros_hbm, acc_sh.at[pl.ds(sid * SLAB + j * ZR, ZR)])

    pltpu.sync_copy(src_hbm.at[pl.ds(base, CH)], sidx0)
    pltpu.sync_copy(dst_hbm.at[pl.ds(base, CH)], didx0)
    pltpu.async_copy(hws_hbm.at[sidx0], rows0, gsem0)
    plsc.subcore_barrier()

    def step(j, carry):
        i0 = 2 * j
        pltpu.sync_copy(src_hbm.at[pl.ds(base + (i0 + 1) * CH, CH)], sidx1)
        pltpu.sync_copy(dst_hbm.at[pl.ds(base + (i0 + 1) * CH, CH)], didx1)
        pltpu.async_copy(hws_hbm.at[sidx1], rows1, gsem1)
        pltpu.make_async_copy(hws_hbm.at[sidx0], rows0, gsem0).wait()
        pltpu.sync_copy(rows0, acc_sh.at[didx0], add=True)

        @pl.when(j < NCHUNK // 2 - 1)
        def _():
            pltpu.sync_copy(src_hbm.at[pl.ds(base + (i0 + 2) * CH, CH)], sidx0)
            pltpu.sync_copy(dst_hbm.at[pl.ds(base + (i0 + 2) * CH, CH)], didx0)
            pltpu.async_copy(hws_hbm.at[sidx0], rows0, gsem0)

        pltpu.make_async_copy(hws_hbm.at[sidx1], rows1, gsem1).wait()
        pltpu.sync_copy(rows1, acc_sh.at[didx1], add=True)
        return carry

    lax.fori_loop(0, NCHUNK // 2, step, 0)
    plsc.subcore_barrier()
    pltpu.sync_copy(acc_sh.at[pl.ds(sid * SLAB, SLAB)],
                    out_hbm.at[cid, pl.ds(sid * SLAB, SLAB)])


@functools.cache
def _build_sc():
    mesh = plsc.VectorSubcoreMesh(core_axis_name="c", subcore_axis_name="s",
                                  num_cores=NC, num_subcores=NS)
    deg_k = pl.kernel(
        _deg_body,
        out_type=jax.ShapeDtypeStruct((NC, NP, H), jnp.float32),
        mesh=mesh,
        scratch_types=[
            pltpu.VMEM((CH,), jnp.int32),
            pltpu.VMEM((CH, H), jnp.float32),
            pltpu.VMEM_SHARED((NP, H), jnp.float32),
        ],
    )
    spmm_k = pl.kernel(
        _spmm_body,
        out_type=jax.ShapeDtypeStruct((NC, NP, H), jnp.float32),
        mesh=mesh,
        scratch_types=[
            pltpu.VMEM((CH,), jnp.int32),
            pltpu.VMEM((CH,), jnp.int32),
            pltpu.VMEM((CH,), jnp.int32),
            pltpu.VMEM((CH,), jnp.int32),
            pltpu.VMEM((CH, H), jnp.float32),
            pltpu.VMEM((CH, H), jnp.float32),
            pltpu.VMEM_SHARED((NP, H), jnp.float32),
            pltpu.SemaphoreType.DMA,
            pltpu.SemaphoreType.DMA,
        ],
    )
    return deg_k, spmm_k


def _sc_degree(dst1, onesH, zerosH):
    return _build_sc()[0](dst1, onesH, zerosH)


def _sc_spmm(src1, dst1, hws, zerosH):
    return _build_sc()[1](src1, dst1, hws, zerosH)


def _dinv_block(deg_ref):
    deg = deg_ref[0, :, 0:1] + deg_ref[1, :, 0:1] + 1.0
    return lax.rsqrt(deg)


def _enc_body(x_ref, deg_ref, sw_ref, sb_ref, tw_ref, tb_ref, w1_ref, o_ref):
    i = pl.program_id(0)
    rowid = lax.broadcasted_iota(jnp.int32, (RB, 1), 0) + i * RB
    is_sig = rowid < NSIG
    w = jnp.where(is_sig, sw_ref[...], tw_ref[...])
    b = jnp.where(is_sig, sb_ref[...], tb_ref[...])
    h0 = jnp.maximum(x_ref[...] * w + b, 0.0)
    hw = jnp.dot(h0, w1_ref[...], preferred_element_type=jnp.float32)
    o_ref[...] = hw * _dinv_block(deg_ref)


def _tc_encode(x, deg, sw, sb, tw, tb, w1):
    return pl.pallas_call(
        _enc_body,
        grid=(GRID,),
        in_specs=[
            pl.BlockSpec((RB, 1), lambda i: (i, 0)),
            pl.BlockSpec((NC, RB, H), lambda i: (0, i, 0)),
            pl.BlockSpec((1, H), lambda i: (0, 0)),
            pl.BlockSpec((1, H), lambda i: (0, 0)),
            pl.BlockSpec((1, H), lambda i: (0, 0)),
            pl.BlockSpec((1, H), lambda i: (0, 0)),
            pl.BlockSpec((H, H), lambda i: (0, 0)),
        ],
        out_specs=pl.BlockSpec((RB, H), lambda i: (i, 0)),
        out_shape=jax.ShapeDtypeStruct((N, H), jnp.float32),
    )(x, deg, sw, sb, tw, tb, w1)


def _mid_body(acc_ref, hws_ref, deg_ref, b1_ref, w2_ref, o_ref):
    dinv = _dinv_block(deg_ref)
    m = acc_ref[0] + acc_ref[1] + hws_ref[...]
    g = jnp.maximum(m * dinv + b1_ref[...], 0.0)
    o_ref[...] = jnp.dot(g, w2_ref[...], preferred_element_type=jnp.float32) * dinv


def _tc_mid(acc, hws, deg, b1, w2):
    return pl.pallas_call(
        _mid_body,
        grid=(GRID,),
        in_specs=[
            pl.BlockSpec((NC, RB, H), lambda i: (0, i, 0)),
            pl.BlockSpec((RB, H), lambda i: (i, 0)),
            pl.BlockSpec((NC, RB, H), lambda i: (0, i, 0)),
            pl.BlockSpec((1, H), lambda i: (0, 0)),
            pl.BlockSpec((H, H), lambda i: (0, 0)),
        ],
        out_specs=pl.BlockSpec((RB, H), lambda i: (i, 0)),
        out_shape=jax.ShapeDtypeStruct((N, H), jnp.float32),
    )(acc, hws, deg, b1, w2)


def _final_body(acc_ref, hws_ref, deg_ref, b2_ref, batch_ref,
                w1_ref, b1_ref, w2_ref, bb2_ref, w3_ref, b3_ref,
                o_ref, sums, counts):
    i = pl.program_id(0)

    @pl.when(i == 0)
    def _():
        sums[...] = jnp.zeros_like(sums)
        counts[...] = jnp.zeros_like(counts)

    dinv = _dinv_block(deg_ref)
    h2 = (acc_ref[0] + acc_ref[1] + hws_ref[...]) * dinv + b2_ref[...]
    oh = (batch_ref[...] == lax.broadcasted_iota(jnp.int32, (RB, G), 1))
    oh = oh.astype(jnp.float32)
    dn = (((0,), (0,)), ((), ()))
    sums[...] += lax.dot_general(oh, h2, dn, preferred_element_type=jnp.float32)
    counts[...] += lax.dot_general(oh, jnp.ones((RB, H), jnp.float32), dn,
                                   preferred_element_type=jnp.float32)

    @pl.when(i == GRID - 1)
    def _():
        pooled = sums[...] / jnp.maximum(counts[...], 1.0)
        z = jnp.maximum(jnp.dot(pooled, w1_ref[...],
                                preferred_element_type=jnp.float32) + b1_ref[...], 0.0)
        z = jnp.maximum(jnp.dot(z, w2_ref[...],
                                preferred_element_type=jnp.float32) + bb2_ref[...], 0.0)
        o_ref[...] = jnp.dot(z, w3_ref[...],
                             preferred_element_type=jnp.float32) + b3_ref[...]


def _tc_final(acc, hws, deg, b2, batch2d, cw1, cb1, cw2, cb2, cw3, cb3):
    return pl.pallas_call(
        _final_body,
        grid=(GRID,),
        in_specs=[
            pl.BlockSpec((NC, RB, H), lambda i: (0, i, 0)),
            pl.BlockSpec((RB, H), lambda i: (i, 0)),
            pl.BlockSpec((NC, RB, H), lambda i: (0, i, 0)),
            pl.BlockSpec((1, H), lambda i: (0, 0)),
            pl.BlockSpec((RB, 1), lambda i: (i, 0)),
            pl.BlockSpec((H, H), lambda i: (0, 0)),
            pl.BlockSpec((1, H), lambda i: (0, 0)),
            pl.BlockSpec((H, H // 2), lambda i: (0, 0)),
            pl.BlockSpec((1, H // 2), lambda i: (0, 0)),
            pl.BlockSpec((H // 2, 4), lambda i: (0, 0)),
            pl.BlockSpec((1, 4), lambda i: (0, 0)),
        ],
        out_specs=pl.BlockSpec((G, 4), lambda i: (0, 0)),
        out_shape=jax.ShapeDtypeStruct((G, 4), jnp.float32),
        scratch_shapes=[
            pltpu.VMEM((G, H), jnp.float32),
            pltpu.VMEM((G, H), jnp.float32),
        ],
    )(acc, hws, deg, b2, batch2d, cw1, cb1, cw2, cb2, cw3, cb3)


def kernel(x, edge_index, batch, sig_W, sig_b, ts_W, ts_b,
           gcn_W1, gcn_b1, gcn_W2, gcn_b2,
           c_W1, c_b1, c_W2, c_b2, c_W3, c_b3):
    npad = EPAD - E
    src1 = jnp.concatenate([edge_index[0], jnp.zeros((npad,), jnp.int32)])
    dst1 = jnp.concatenate([edge_index[1], jnp.full((npad,), DUMP, jnp.int32)])
    onesH = jnp.ones((CH, H), jnp.float32)
    zerosH = jnp.zeros((ZR, H), jnp.float32)

    deg = _sc_degree(dst1, onesH, zerosH)
    hws1 = _tc_encode(x, deg, sig_W.reshape(1, H), sig_b.reshape(1, H),
                      ts_W.reshape(1, H), ts_b.reshape(1, H), gcn_W1)
    acc1 = _sc_spmm(src1, dst1, hws1, zerosH)
    hws2 = _tc_mid(acc1, hws1, deg, gcn_b1.reshape(1, H), gcn_W2)
    acc2 = _sc_spmm(src1, dst1, hws2, zerosH)
    return _tc_final(acc2, hws2, deg, gcn_b2.reshape(1, H),
                     batch.reshape(N, 1), c_W1, c_b1.reshape(1, H),
                     c_W2, c_b2.reshape(1, H // 2), c_W3, c_b3.reshape(1, 4))
